# baseline (speedup 1.0000x reference)
import functools

import jax
import jax.numpy as jnp
from jax import lax
from jax.experimental import pallas as pl
from jax.experimental.pallas import tpu as pltpu
from jax.experimental.pallas import tpu_sc as plsc

NP = 640
D = 64
K = 5
NQ = 100000
BQ = 4000
NB = NQ // BQ

_NEG = float("-inf")
_BIGI = 2**30


def _topk_body(p_ref, q_ref, out_ref, rv_ref, ri_ref):
    j = pl.program_id(0)

    @pl.when(j == 0)
    def _init():
        rv_ref[...] = jnp.full((NP, 8), _NEG, jnp.float32)
        ri_ref[...] = jnp.zeros((NP, 8), jnp.int32)

    p = p_ref[...]
    pn = p / jnp.sqrt(jnp.sum(p * p, axis=1, keepdims=True))
    q = q_ref[...]
    qn = q / jnp.sqrt(jnp.sum(q * q, axis=1, keepdims=True))
    s = lax.dot_general(pn, qn, (((1,), (1,)), ((), ())),
                        preferred_element_type=jnp.float32)

    iota = lax.broadcasted_iota(jnp.int32, (NP, BQ), 1).astype(jnp.float32)
    base = j * BQ
    bv, bi = [], []
    for t in range(K):
        m = jnp.max(s, axis=1, keepdims=True)
        c = jnp.where(s == m, iota, jnp.inf)
        li = jnp.min(c, axis=1, keepdims=True)
        bv.append(m)
        bi.append(li.astype(jnp.int32) + base)
        if t != K - 1:
            s = jnp.where(iota == li, _NEG, s)

    av = jnp.concatenate([rv_ref[:, :K]] + bv, axis=1)
    ai = jnp.concatenate([ri_ref[:, :K]] + bi, axis=1)
    iota10 = lax.broadcasted_iota(jnp.int32, (NP, 2 * K), 1)
    nv, ni = [], []
    for t in range(K):
        m = jnp.max(av, axis=1, keepdims=True)
        c = jnp.where(av == m, iota10, _BIGI)
        sel = jnp.min(c, axis=1, keepdims=True)
        pick = iota10 == sel
        ni.append(jnp.sum(jnp.where(pick, ai, 0), axis=1, keepdims=True))
        nv.append(m)
        if t != K - 1:
            av = jnp.where(pick, _NEG, av)
    pad_v = jnp.full((NP, 8 - K), _NEG, jnp.float32)
    pad_i = jnp.zeros((NP, 8 - K), jnp.int32)
    rv_ref[...] = jnp.concatenate(nv + [pad_v], axis=1)
    ri_ref[...] = jnp.concatenate(ni + [pad_i], axis=1)
    out_ref[...] = ri_ref[...]


def _topk_tc(pflat, queries):
    return pl.pallas_call(
        _topk_body,
        grid=(NB,),
        in_specs=[
            pl.BlockSpec((NP, D), lambda j: (0, 0)),
            pl.BlockSpec((BQ, D), lambda j: (j, 0)),
        ],
        out_specs=pl.BlockSpec((NP, 8), lambda j: (0, 0)),
        out_shape=jax.ShapeDtypeStruct((NP, 8), jnp.int32),
        scratch_shapes=[
            pltpu.VMEM((NP, 8), jnp.float32),
            pltpu.VMEM((NP, 8), jnp.int32),
        ],
        compiler_params=pltpu.CompilerParams(
            dimension_semantics=("arbitrary",),
            vmem_limit_bytes=128 * 1024 * 1024,
        ),
    )(pflat, queries)


_SC_B = 3328
_NW = 32
_BPW = _SC_B // _NW


def _gather_body(table_hbm, idx_hbm, out_hbm, idx_v, rows_v, sem):
    wid = lax.axis_index("s") * 2 + lax.axis_index("c")
    base = wid * _BPW
    pltpu.sync_copy(idx_hbm.at[pl.ds(base, _BPW)], idx_v)
    pltpu.async_copy(table_hbm.at[idx_v], rows_v, sem).wait()
    pltpu.sync_copy(rows_v, out_hbm.at[pl.ds(base, _BPW)])


def _gather_sc(queries, idx_pad):
    mesh = plsc.VectorSubcoreMesh(core_axis_name="c", subcore_axis_name="s")
    f = functools.partial(
        pl.kernel,
        mesh=mesh,
        out_type=jax.ShapeDtypeStruct((_SC_B, D), jnp.float32),
        scratch_types=[
            pltpu.VMEM((_BPW,), jnp.int32),
            pltpu.VMEM((_BPW, D), jnp.float32),
            pltpu.SemaphoreType.DMA,
        ],
        compiler_params=pltpu.CompilerParams(use_tc_tiling_on_sc=False),
    )(_gather_body)
    return f(queries, idx_pad)


def kernel(prototypes, queries):
    nway, kshot, dim = prototypes.shape
    idx8 = _topk_tc(prototypes.reshape(nway * kshot, dim), queries)
    nidx = idx8[:, :K].reshape(-1)
    idx_pad = jnp.concatenate(
        [nidx, jnp.zeros((_SC_B - nidx.shape[0],), jnp.int32)])
    rows = _gather_sc(queries, idx_pad)
    return rows[: nidx.shape[0]].reshape(nway, kshot * K, dim)


# speedup vs baseline: 1.0320x; 1.0320x over previous
import functools

import jax
import jax.numpy as jnp
from jax import lax
from jax.experimental import pallas as pl
from jax.experimental.pallas import tpu as pltpu
from jax.experimental.pallas import tpu_sc as plsc

NP = 640
D = 64
K = 5
NQ = 100000
BQ = 5000
NB = NQ // BQ

_NEG = float("-inf")
_BIGI = 2**30


def _topk_body(p_ref, q_ref, out_ref, rv_ref, ri_ref):
    j = pl.program_id(0)

    @pl.when(j == 0)
    def _init():
        rv_ref[...] = jnp.full((NP, 8), _NEG, jnp.float32)
        ri_ref[...] = jnp.zeros((NP, 8), jnp.int32)

    p = p_ref[...]
    pn = p / jnp.sqrt(jnp.sum(p * p, axis=1, keepdims=True))
    q = q_ref[...]
    qn = q / jnp.sqrt(jnp.sum(q * q, axis=1, keepdims=True))
    s = lax.dot_general(pn, qn, (((1,), (1,)), ((), ())),
                        preferred_element_type=jnp.float32)

    iota = lax.broadcasted_iota(jnp.int32, (NP, BQ), 1).astype(jnp.float32)
    base = j * BQ
    bv, bi = [], []
    for t in range(K):
        m = jnp.max(s, axis=1, keepdims=True)
        c = jnp.where(s == m, iota, jnp.inf)
        li = jnp.min(c, axis=1, keepdims=True)
        bv.append(m)
        bi.append(li.astype(jnp.int32) + base)
        if t != K - 1:
            s = jnp.where(iota == li, _NEG, s)

    av = jnp.concatenate([rv_ref[:, :K]] + bv, axis=1)
    ai = jnp.concatenate([ri_ref[:, :K]] + bi, axis=1)
    iota10 = lax.broadcasted_iota(jnp.int32, (NP, 2 * K), 1)
    nv, ni = [], []
    for t in range(K):
        m = jnp.max(av, axis=1, keepdims=True)
        c = jnp.where(av == m, iota10, _BIGI)
        sel = jnp.min(c, axis=1, keepdims=True)
        pick = iota10 == sel
        ni.append(jnp.sum(jnp.where(pick, ai, 0), axis=1, keepdims=True))
        nv.append(m)
        if t != K - 1:
            av = jnp.where(pick, _NEG, av)
    pad_v = jnp.full((NP, 8 - K), _NEG, jnp.float32)
    pad_i = jnp.zeros((NP, 8 - K), jnp.int32)
    rv_ref[...] = jnp.concatenate(nv + [pad_v], axis=1)
    ri_ref[...] = jnp.concatenate(ni + [pad_i], axis=1)
    out_ref[...] = ri_ref[...]


def _topk_tc(pflat, queries):
    return pl.pallas_call(
        _topk_body,
        grid=(NB,),
        in_specs=[
            pl.BlockSpec((NP, D), lambda j: (0, 0)),
            pl.BlockSpec((BQ, D), lambda j: (j, 0)),
        ],
        out_specs=pl.BlockSpec((NP, 8), lambda j: (0, 0)),
        out_shape=jax.ShapeDtypeStruct((NP, 8), jnp.int32),
        scratch_shapes=[
            pltpu.VMEM((NP, 8), jnp.float32),
            pltpu.VMEM((NP, 8), jnp.int32),
        ],
        compiler_params=pltpu.CompilerParams(
            dimension_semantics=("arbitrary",),
            vmem_limit_bytes=128 * 1024 * 1024,
        ),
    )(pflat, queries)


_SC_B = 3328
_NW = 32
_BPW = _SC_B // _NW


def _gather_body(table_hbm, idx_hbm, out_hbm, idx_v, rows_v, sem):
    wid = lax.axis_index("s") * 2 + lax.axis_index("c")
    base = wid * _BPW
    pltpu.sync_copy(idx_hbm.at[pl.ds(base, _BPW)], idx_v)
    pltpu.async_copy(table_hbm.at[idx_v], rows_v, sem).wait()
    pltpu.sync_copy(rows_v, out_hbm.at[pl.ds(base, _BPW)])


def _gather_sc(queries, idx_pad):
    mesh = plsc.VectorSubcoreMesh(core_axis_name="c", subcore_axis_name="s")
    f = functools.partial(
        pl.kernel,
        mesh=mesh,
        out_type=jax.ShapeDtypeStruct((_SC_B, D), jnp.float32),
        scratch_types=[
            pltpu.VMEM((_BPW,), jnp.int32),
            pltpu.VMEM((_BPW, D), jnp.float32),
            pltpu.SemaphoreType.DMA,
        ],
        compiler_params=pltpu.CompilerParams(use_tc_tiling_on_sc=False),
    )(_gather_body)
    return f(queries, idx_pad)


def kernel(prototypes, queries):
    nway, kshot, dim = prototypes.shape
    idx8 = _topk_tc(prototypes.reshape(nway * kshot, dim), queries)
    nidx = idx8[:, :K].reshape(-1)
    idx_pad = jnp.concatenate(
        [nidx, jnp.zeros((_SC_B - nidx.shape[0],), jnp.int32)])
    rows = _gather_sc(queries, idx_pad)
    return rows[: nidx.shape[0]].reshape(nway, kshot * K, dim)
